# baseline (speedup 1.0000x reference)
import functools
import jax
import jax.numpy as jnp
from jax.experimental import pallas as pl
from jax.experimental.pallas import tpu as pltpu
from jax.experimental.pallas import tpu_sc as plsc

BLK = 256
RCH = 512
SCW = 128


def _route_kernel(x_ref, wg_ref, w12_ref, s12_ref, be_ref,
                  nblk_ref, xlo_ref, xhi_ref, sel_ref, i1_ref, i2_ref,
                  ltri_ref, *, n, e_num, nb, blk):
    d2 = x_ref.shape[1] // 2
    rch = RCH

    r0 = jax.lax.broadcasted_iota(jnp.int32, (rch, rch), 0)
    r1 = jax.lax.broadcasted_iota(jnp.int32, (rch, rch), 1)
    ltri_ref[...] = jnp.where(r0 > r1, 1.0, 0.0).astype(jnp.bfloat16)

    def p1(r, counts):
        sl = pl.ds(r * rch, rch)
        lg = jax.lax.dot_general(
            x_ref[sl, :], wg_ref[...], (((1,), (1,)), ((), ())),
            preferred_element_type=jnp.float32)
        xlo_ref[sl, :] = x_ref[sl, 0:d2]
        xhi_ref[sl, :] = x_ref[sl, d2:2 * d2]
        iota = jax.lax.broadcasted_iota(jnp.int32, lg.shape, 1)
        m1 = jnp.max(lg, axis=1, keepdims=True)
        i1 = jnp.min(jnp.where(lg == m1, iota, e_num), axis=1, keepdims=True)
        l2 = jnp.where(iota == i1, -jnp.inf, lg)
        m2 = jnp.max(l2, axis=1, keepdims=True)
        i2 = jnp.min(jnp.where(l2 == m2, iota, e_num), axis=1, keepdims=True)
        z = jnp.exp(m2 - m1)
        w12_ref[sl, :] = jnp.where(iota == 0, 1.0 / (1.0 + z), z / (1.0 + z))
        sel = (jnp.where(iota == i1, 1.0, 0.0)
               + jnp.where(iota == i2, 1.0, 0.0))
        sel_ref[sl, :] = sel.astype(jnp.bfloat16)
        i1_ref[sl, :] = i1
        i2_ref[sl, :] = i2
        return counts + jnp.sum(sel, axis=0, keepdims=True)

    counts = jax.lax.fori_loop(
        0, n // rch, p1, jnp.zeros((1, e_num), jnp.float32))

    pc = jnp.ceil(counts * (1.0 / blk)) * blk
    pc8 = jnp.broadcast_to(pc, (e_num, e_num))
    i0 = jax.lax.broadcasted_iota(jnp.int32, (e_num, 128), 0)
    i1g = jax.lax.broadcasted_iota(jnp.int32, (e_num, 128), 1)
    upper = jnp.where(i0 < i1g, 1.0, 0.0)
    base128 = jax.lax.dot_general(
        pc8, upper, (((1,), (0,)), ((), ())),
        preferred_element_type=jnp.float32)
    base_row = base128[0:1, 0:e_num]
    c_row = base_row + pc

    nblk_ref[...] = (pc * (1.0 / blk)).astype(jnp.int32)
    c_nb = jnp.broadcast_to(c_row, (nb, e_num))
    b_iota = (jax.lax.broadcasted_iota(jnp.int32, (nb, e_num), 0)
              * blk).astype(jnp.float32)
    be = jnp.sum(jnp.where(c_nb <= b_iota, 1.0, 0.0), axis=1, keepdims=True)
    be_ref[...] = jnp.minimum(be, e_num - 1).astype(jnp.int32)

    def p2(r, run):
        sl = pl.ds(r * rch, rch)
        sel = sel_ref[sl, :]
        i1 = i1_ref[sl, :]
        i2 = i2_ref[sl, :]
        iota = jax.lax.broadcasted_iota(jnp.int32, (rch, e_num), 1)
        pos = jax.lax.dot_general(
            ltri_ref[...], sel, (((1,), (0,)), ((), ())),
            preferred_element_type=jnp.float32) + run
        base_b = jnp.broadcast_to(base_row, (rch, e_num))
        pos1 = jnp.sum(jnp.where(iota == i1, pos + base_b, 0.0),
                       axis=1, keepdims=True)
        pos2 = jnp.sum(jnp.where(iota == i2, pos + base_b, 0.0),
                       axis=1, keepdims=True)
        s1v = pos1.astype(jnp.int32)
        s2v = pos2.astype(jnp.int32)
        p_tot = nb * blk
        for q in range(6):
            s12_ref[pl.ds(q * 2 * n + r * rch, rch), :] = s1v + q * p_tot
            s12_ref[pl.ds(q * 2 * n + n + r * rch, rch), :] = s2v + q * p_tot
        return run + jnp.sum(sel.astype(jnp.float32), axis=0, keepdims=True)

    jax.lax.fori_loop(0, n // rch, p2, jnp.zeros((1, e_num), jnp.float32))


def _route(x, Wg, nb):
    n, d = x.shape
    e_num = Wg.shape[0]
    kern = functools.partial(_route_kernel, n=n, e_num=e_num, nb=nb, blk=BLK)
    return pl.pallas_call(
        kern,
        grid=(1,),
        in_specs=[
            pl.BlockSpec((n, d), lambda i: (0, 0)),
            pl.BlockSpec((e_num, d), lambda i: (0, 0)),
        ],
        out_specs=[
            pl.BlockSpec((n, e_num), lambda i: (0, 0)),
            pl.BlockSpec((12 * n, 1), lambda i: (0, 0)),
            pl.BlockSpec((nb, 1), lambda i: (0, 0)),
            pl.BlockSpec((1, e_num), lambda i: (0, 0)),
            pl.BlockSpec((n, d // 2), lambda i: (0, 0)),
            pl.BlockSpec((n, d // 2), lambda i: (0, 0)),
        ],
        out_shape=[
            jax.ShapeDtypeStruct((n, e_num), jnp.float32),
            jax.ShapeDtypeStruct((12 * n, 1), jnp.int32),
            jax.ShapeDtypeStruct((nb, 1), jnp.int32),
            jax.ShapeDtypeStruct((1, e_num), jnp.int32),
            jax.ShapeDtypeStruct((n, d // 2), jnp.float32),
            jax.ShapeDtypeStruct((n, d // 2), jnp.float32),
        ],
        scratch_shapes=[
            pltpu.VMEM((n, e_num), jnp.bfloat16),
            pltpu.VMEM((n, 1), jnp.int32),
            pltpu.VMEM((n, 1), jnp.int32),
            pltpu.VMEM((RCH, RCH), jnp.bfloat16),
        ],
    )(x, Wg)


def _sc_dispatch(xlo, xhi, s12r, p):
    n2, d2 = 2 * xlo.shape[0], xlo.shape[1]
    nblks = xlo.shape[0] // SCW
    mesh = plsc.VectorSubcoreMesh(core_axis_name="c", subcore_axis_name="s")
    out_type = [jax.ShapeDtypeStruct((p, d2), jnp.float32)] * 2

    @functools.partial(pl.kernel, out_type=out_type, mesh=mesh)
    def k(xlo_hbm, xhi_hbm, s_hbm, olo_hbm, ohi_hbm):
        for x_hbm, o_hbm in ((xlo_hbm, olo_hbm), (xhi_hbm, ohi_hbm)):
            def body(x_vmem, i_vmem, o_hbm=o_hbm):
                pltpu.sync_copy(x_vmem, o_hbm.at[i_vmem.at[0]])

            pltpu.emit_pipeline(
                body,
                grid=(n2 // SCW,),
                in_specs=[
                    pl.BlockSpec((SCW, d2), lambda i: (i % nblks, 0)),
                    pl.BlockSpec((1, SCW), lambda i: (0, i)),
                ],
                out_specs=[],
                core_axis_name=("c", "s"),
                dimension_semantics=(pltpu.PARALLEL,),
            )(x_hbm, s_hbm)

    return k(xlo, xhi, s12r)


def _mlp_kernel(be_ref, xlo_ref, xhi_ref, w1_ref, w2_ref, b1_ref, b2_ref,
                yst_ref, yscr_ref):
    d6 = yst_ref.shape[1]
    q = pl.program_id(1)

    @pl.when(q == 0)
    def _():
        xb = jnp.concatenate([xlo_ref[...], xhi_ref[...]],
                             axis=1).astype(jnp.bfloat16)
        h = jax.lax.dot_general(
            xb, w1_ref[0], (((1,), (1,)), ((), ())),
            preferred_element_type=jnp.float32)
        h = h + b1_ref[0]
        h = 0.5 * h * (1.0 + jax.lax.erf(h * 0.7071067811865476))
        hb = h.astype(jnp.bfloat16)
        y = jax.lax.dot_general(
            hb, w2_ref[0], (((1,), (1,)), ((), ())),
            preferred_element_type=jnp.float32)
        yscr_ref[...] = y + b2_ref[0]

    yst_ref[...] = yscr_ref[:, pl.ds(q * d6, d6)]


def _grouped_mlp(be, xslo, xshi, w1b, w2b, b1, b2, nb, p):
    d, f = w2b.shape[1], w2b.shape[2]
    d2 = d // 2
    d6 = d // 6
    e_num = w1b.shape[0]
    grid_spec = pltpu.PrefetchScalarGridSpec(
        num_scalar_prefetch=1,
        grid=(nb, 6),
        in_specs=[
            pl.BlockSpec((BLK, d2), lambda b, q, be_r: (b, 0)),
            pl.BlockSpec((BLK, d2), lambda b, q, be_r: (b, 0)),
            pl.BlockSpec((1, f, d), lambda b, q, be_r: (be_r[b], 0, 0)),
            pl.BlockSpec((1, d, f), lambda b, q, be_r: (be_r[b], 0, 0)),
            pl.BlockSpec((1, 1, f), lambda b, q, be_r: (be_r[b], 0, 0)),
            pl.BlockSpec((1, 1, d), lambda b, q, be_r: (be_r[b], 0, 0)),
        ],
        out_specs=pl.BlockSpec(
            (BLK, d6), lambda b, q, be_r: (q * (p // BLK) + b, 0)),
        scratch_shapes=[pltpu.VMEM((BLK, d), jnp.float32)],
    )
    return pl.pallas_call(
        _mlp_kernel,
        grid_spec=grid_spec,
        out_shape=jax.ShapeDtypeStruct((6 * p, d6), jnp.float32),
        compiler_params=pltpu.CompilerParams(
            dimension_semantics=("arbitrary", "arbitrary"),
            vmem_limit_bytes=100 * 1024 * 1024,
        ),
    )(be, xslo, xshi, w1b, w2b,
      b1.reshape(e_num, 1, f), b2.reshape(e_num, 1, d))


def _sc_combine(yst, s_trip, w12, n, d):
    d6 = yst.shape[1]
    nwin = n // SCW
    mesh = plsc.VectorSubcoreMesh(core_axis_name="c", subcore_axis_name="s")

    @functools.partial(
        pl.kernel,
        out_type=jax.ShapeDtypeStruct((n, d), jnp.float32),
        mesh=mesh,
        scratch_types=[pltpu.VMEM((SCW, d6), jnp.float32)])
    def k(y_hbm, s_hbm, w_hbm, o_hbm, g0):
        def body(s1_vmem, s2_vmem, w_vmem, o_vmem):
            pltpu.sync_copy(y_hbm.at[s1_vmem.at[0]], o_vmem)
            pltpu.sync_copy(y_hbm.at[s2_vmem.at[0]], g0)
            wa = w_vmem[:, 0:1]
            wb = w_vmem[:, 1:2]
            o_vmem[...] = wa * o_vmem[...] + wb * g0[...]

        pltpu.emit_pipeline(
            body,
            grid=(6 * nwin,),
            in_specs=[
                pl.BlockSpec(
                    (1, SCW),
                    lambda i: (0, (i // nwin) * 2 * nwin + i % nwin)),
                pl.BlockSpec(
                    (1, SCW),
                    lambda i: (0, (i // nwin) * 2 * nwin + nwin + i % nwin)),
                pl.BlockSpec((SCW, 8), lambda i: (i % nwin, 0)),
            ],
            out_specs=[
                pl.BlockSpec((SCW, d6), lambda i: (i % nwin, i // nwin)),
            ],
            core_axis_name=("c", "s"),
            dimension_semantics=(pltpu.PARALLEL,),
        )(s_hbm, s_hbm, w_hbm, o_hbm)

    return k(yst, s_trip, w12)


def kernel(x, Wg, W1, b1, W2, b2):
    bv, tv, d = x.shape
    n = bv * tv
    e_num, f = W1.shape[0], W1.shape[1]
    nb = 2 * n // BLK + e_num
    p = nb * BLK

    xf = x.reshape(n, d)
    w1b = W1.astype(jnp.bfloat16)
    w2b = W2.astype(jnp.bfloat16)

    w12, s_trip, be, nblk, xlo, xhi = _route(xf, Wg, nb)
    s_tripr = s_trip.reshape(1, 12 * n)

    xslo, xshi = _sc_dispatch(xlo, xhi, s_tripr, p)
    yst = _grouped_mlp(be.reshape(nb), xslo, xshi, w1b, w2b, b1, b2, nb, p)
    out = _sc_combine(yst, s_tripr, w12, n, d)
    return out.reshape(bv, tv, d)


# speedup vs baseline: 1.6571x; 1.6571x over previous
import functools
import jax
import jax.numpy as jnp
from jax.experimental import pallas as pl
from jax.experimental.pallas import tpu as pltpu
from jax.experimental.pallas import tpu_sc as plsc

BLK = 256
RCH = 512
SCW = 128


def _route_kernel(x_ref, wg_ref, w1_ref, w2_ref, s12_ref, be_ref,
                  nblk_ref, xlo_ref, xhi_ref, sel_ref, i1_ref, i2_ref,
                  ltri_ref, *, n, e_num, nb, blk):
    d2 = x_ref.shape[1] // 2
    rch = RCH

    r0 = jax.lax.broadcasted_iota(jnp.int32, (rch, rch), 0)
    r1 = jax.lax.broadcasted_iota(jnp.int32, (rch, rch), 1)
    ltri_ref[...] = jnp.where(r0 > r1, 1.0, 0.0).astype(jnp.bfloat16)

    def p1(r, counts):
        sl = pl.ds(r * rch, rch)
        lg = jax.lax.dot_general(
            x_ref[sl, :], wg_ref[...], (((1,), (1,)), ((), ())),
            preferred_element_type=jnp.float32)
        xlo_ref[sl, :] = x_ref[sl, 0:d2]
        xhi_ref[sl, :] = x_ref[sl, d2:2 * d2]
        iota = jax.lax.broadcasted_iota(jnp.int32, lg.shape, 1)
        m1 = jnp.max(lg, axis=1, keepdims=True)
        i1 = jnp.min(jnp.where(lg == m1, iota, e_num), axis=1, keepdims=True)
        l2 = jnp.where(iota == i1, -jnp.inf, lg)
        m2 = jnp.max(l2, axis=1, keepdims=True)
        i2 = jnp.min(jnp.where(l2 == m2, iota, e_num), axis=1, keepdims=True)
        z = jnp.exp(m2 - m1)
        w1_ref[sl, :] = 1.0 / (1.0 + z)
        w2_ref[sl, :] = z / (1.0 + z)
        sel = (jnp.where(iota == i1, 1.0, 0.0)
               + jnp.where(iota == i2, 1.0, 0.0))
        sel_ref[sl, :] = sel.astype(jnp.bfloat16)
        i1_ref[sl, :] = i1
        i2_ref[sl, :] = i2
        return counts + jnp.sum(sel, axis=0, keepdims=True)

    counts = jax.lax.fori_loop(
        0, n // rch, p1, jnp.zeros((1, e_num), jnp.float32))

    pc = jnp.ceil(counts * (1.0 / blk)) * blk
    pc8 = jnp.broadcast_to(pc, (e_num, e_num))
    i0 = jax.lax.broadcasted_iota(jnp.int32, (e_num, 128), 0)
    i1g = jax.lax.broadcasted_iota(jnp.int32, (e_num, 128), 1)
    upper = jnp.where(i0 < i1g, 1.0, 0.0)
    base128 = jax.lax.dot_general(
        pc8, upper, (((1,), (0,)), ((), ())),
        preferred_element_type=jnp.float32)
    base_row = base128[0:1, 0:e_num]
    c_row = base_row + pc

    nblk_ref[...] = (pc * (1.0 / blk)).astype(jnp.int32)
    c_nb = jnp.broadcast_to(c_row, (nb, e_num))
    b_iota = (jax.lax.broadcasted_iota(jnp.int32, (nb, e_num), 0)
              * blk).astype(jnp.float32)
    be = jnp.sum(jnp.where(c_nb <= b_iota, 1.0, 0.0), axis=1, keepdims=True)
    be_ref[...] = jnp.minimum(be, e_num - 1).astype(jnp.int32)

    def p2(r, run):
        sl = pl.ds(r * rch, rch)
        sel = sel_ref[sl, :]
        i1 = i1_ref[sl, :]
        i2 = i2_ref[sl, :]
        iota = jax.lax.broadcasted_iota(jnp.int32, (rch, e_num), 1)
        pos = jax.lax.dot_general(
            ltri_ref[...], sel, (((1,), (0,)), ((), ())),
            preferred_element_type=jnp.float32) + run
        base_b = jnp.broadcast_to(base_row, (rch, e_num))
        pos1 = jnp.sum(jnp.where(iota == i1, pos + base_b, 0.0),
                       axis=1, keepdims=True)
        pos2 = jnp.sum(jnp.where(iota == i2, pos + base_b, 0.0),
                       axis=1, keepdims=True)
        s12_ref[sl, :] = pos1.astype(jnp.int32)
        s12_ref[pl.ds(n + r * rch, rch), :] = pos2.astype(jnp.int32)
        return run + jnp.sum(sel.astype(jnp.float32), axis=0, keepdims=True)

    jax.lax.fori_loop(0, n // rch, p2, jnp.zeros((1, e_num), jnp.float32))


def _route(x, Wg, nb):
    n, d = x.shape
    e_num = Wg.shape[0]
    kern = functools.partial(_route_kernel, n=n, e_num=e_num, nb=nb, blk=BLK)
    return pl.pallas_call(
        kern,
        grid=(1,),
        in_specs=[
            pl.BlockSpec((n, d), lambda i: (0, 0)),
            pl.BlockSpec((e_num, d), lambda i: (0, 0)),
        ],
        out_specs=[
            pl.BlockSpec((n, 1), lambda i: (0, 0)),
            pl.BlockSpec((n, 1), lambda i: (0, 0)),
            pl.BlockSpec((2 * n, 1), lambda i: (0, 0)),
            pl.BlockSpec((nb, 1), lambda i: (0, 0)),
            pl.BlockSpec((1, e_num), lambda i: (0, 0)),
            pl.BlockSpec((n, d // 2), lambda i: (0, 0)),
            pl.BlockSpec((n, d // 2), lambda i: (0, 0)),
        ],
        out_shape=[
            jax.ShapeDtypeStruct((n, 1), jnp.float32),
            jax.ShapeDtypeStruct((n, 1), jnp.float32),
            jax.ShapeDtypeStruct((2 * n, 1), jnp.int32),
            jax.ShapeDtypeStruct((nb, 1), jnp.int32),
            jax.ShapeDtypeStruct((1, e_num), jnp.int32),
            jax.ShapeDtypeStruct((n, d // 2), jnp.float32),
            jax.ShapeDtypeStruct((n, d // 2), jnp.float32),
        ],
        scratch_shapes=[
            pltpu.VMEM((n, e_num), jnp.bfloat16),
            pltpu.VMEM((n, 1), jnp.int32),
            pltpu.VMEM((n, 1), jnp.int32),
            pltpu.VMEM((RCH, RCH), jnp.bfloat16),
        ],
    )(x, Wg)


def _sc_dispatch(xlo, xhi, s12r, p):
    n2, d2 = 2 * xlo.shape[0], xlo.shape[1]
    nblks = xlo.shape[0] // SCW
    mesh = plsc.VectorSubcoreMesh(core_axis_name="c", subcore_axis_name="s")
    out_type = [jax.ShapeDtypeStruct((p, d2), jnp.float32)] * 2

    @functools.partial(pl.kernel, out_type=out_type, mesh=mesh)
    def k(xlo_hbm, xhi_hbm, s_hbm, olo_hbm, ohi_hbm):
        for x_hbm, o_hbm in ((xlo_hbm, olo_hbm), (xhi_hbm, ohi_hbm)):
            def body(x_vmem, i_vmem, o_hbm=o_hbm):
                pltpu.sync_copy(x_vmem, o_hbm.at[i_vmem.at[0]])

            pltpu.emit_pipeline(
                body,
                grid=(n2 // SCW,),
                in_specs=[
                    pl.BlockSpec((SCW, d2), lambda i: (i % nblks, 0)),
                    pl.BlockSpec((1, SCW), lambda i: (0, i)),
                ],
                out_specs=[],
                core_axis_name=("c", "s"),
                dimension_semantics=(pltpu.PARALLEL,),
            )(x_hbm, s_hbm)

    return k(xlo, xhi, s12r)


def _mlp_kernel(be_ref, nblk_ref, xlo_ref, xhi_ref, w1_ref, w2_ref, b1_ref,
                b2_ref, ylo_ref, yhi_ref):
    d2 = xlo_ref.shape[1]
    b = pl.program_id(0)
    tot = sum(nblk_ref[i] for i in range(nblk_ref.shape[0]))

    @pl.when(b < tot)
    def _():
        xb = jnp.concatenate([xlo_ref[...], xhi_ref[...]],
                             axis=1).astype(jnp.bfloat16)
        h = jax.lax.dot_general(
            xb, w1_ref[0], (((1,), (1,)), ((), ())),
            preferred_element_type=jnp.float32)
        h = h + b1_ref[0]
        h = 0.5 * h * (1.0 + jax.lax.erf(h * 0.7071067811865476))
        hb = h.astype(jnp.bfloat16)
        y = jax.lax.dot_general(
            hb, w2_ref[0], (((1,), (1,)), ((), ())),
            preferred_element_type=jnp.float32)
        y = y + b2_ref[0]
        ylo_ref[...] = y[:, 0:d2]
        yhi_ref[...] = y[:, d2:2 * d2]


def _grouped_mlp(be, nblk8, xslo, xshi, w1b, w2b, b1, b2, nb, p):
    d, f = w2b.shape[1], w2b.shape[2]
    d2 = d // 2
    e_num = w1b.shape[0]
    grid_spec = pltpu.PrefetchScalarGridSpec(
        num_scalar_prefetch=2,
        grid=(nb,),
        in_specs=[
            pl.BlockSpec((BLK, d2), lambda b, be_r, nb_r: (b, 0)),
            pl.BlockSpec((BLK, d2), lambda b, be_r, nb_r: (b, 0)),
            pl.BlockSpec((1, f, d), lambda b, be_r, nb_r: (be_r[b], 0, 0)),
            pl.BlockSpec((1, d, f), lambda b, be_r, nb_r: (be_r[b], 0, 0)),
            pl.BlockSpec((1, 1, f), lambda b, be_r, nb_r: (be_r[b], 0, 0)),
            pl.BlockSpec((1, 1, d), lambda b, be_r, nb_r: (be_r[b], 0, 0)),
        ],
        out_specs=[
            pl.BlockSpec((BLK, d2), lambda b, be_r, nb_r: (b, 0)),
            pl.BlockSpec((BLK, d2), lambda b, be_r, nb_r: (b, 0)),
        ],
    )
    return pl.pallas_call(
        _mlp_kernel,
        grid_spec=grid_spec,
        out_shape=[jax.ShapeDtypeStruct((p, d2), jnp.float32)] * 2,
        compiler_params=pltpu.CompilerParams(
            dimension_semantics=("arbitrary",),
            vmem_limit_bytes=100 * 1024 * 1024,
        ),
    )(be, nblk8, xslo, xshi, w1b, w2b,
      b1.reshape(e_num, 1, f), b2.reshape(e_num, 1, d))


def _sc_combine(ylo, yhi, s12r, n):
    d2 = ylo.shape[1]
    mesh = plsc.VectorSubcoreMesh(core_axis_name="c", subcore_axis_name="s")
    out_type = [jax.ShapeDtypeStruct((2 * n, d2), jnp.float32)] * 2

    @functools.partial(pl.kernel, out_type=out_type, mesh=mesh)
    def k(ylo_hbm, yhi_hbm, s_hbm, glo_hbm, ghi_hbm):
        for y_hbm, g_hbm in ((ylo_hbm, glo_hbm), (yhi_hbm, ghi_hbm)):
            def body(i_vmem, o_vmem, y_hbm=y_hbm):
                pltpu.sync_copy(y_hbm.at[i_vmem.at[0]], o_vmem)

            pltpu.emit_pipeline(
                body,
                grid=(2 * n // SCW,),
                in_specs=[pl.BlockSpec((1, SCW), lambda i: (0, i))],
                out_specs=[pl.BlockSpec((SCW, d2), lambda i: (i, 0))],
                core_axis_name=("c", "s"),
                dimension_semantics=(pltpu.PARALLEL,),
            )(s_hbm, g_hbm)

    return k(ylo, yhi, s12r)


def _comb_kernel(w1_ref, w2_ref, y0lo_ref, y0hi_ref, y1lo_ref, y1hi_ref,
                 o_ref):
    d2 = y0lo_ref.shape[1]
    w1 = w1_ref[...]
    w2 = w2_ref[...]
    o_ref[:, 0:d2] = w1 * y0lo_ref[...] + w2 * y1lo_ref[...]
    o_ref[:, d2:2 * d2] = w1 * y0hi_ref[...] + w2 * y1hi_ref[...]


def _combine(w1v, w2v, glo, ghi):
    n2, d2 = glo.shape
    n = n2 // 2
    cch = 1024
    nch = n // cch
    top = pl.BlockSpec((cch, d2), lambda i: (i, 0))
    bot = pl.BlockSpec((cch, d2), lambda i: (i + nch, 0))
    col = pl.BlockSpec((cch, 1), lambda i: (i, 0))
    return pl.pallas_call(
        _comb_kernel,
        grid=(nch,),
        in_specs=[col, col, top, top, bot, bot],
        out_specs=pl.BlockSpec((cch, 2 * d2), lambda i: (i, 0)),
        out_shape=jax.ShapeDtypeStruct((n, 2 * d2), jnp.float32),
    )(w1v, w2v, glo, ghi, glo, ghi)


def kernel(x, Wg, W1, b1, W2, b2):
    bv, tv, d = x.shape
    n = bv * tv
    e_num, f = W1.shape[0], W1.shape[1]
    nb = 2 * n // BLK + e_num
    p = nb * BLK

    xf = x.reshape(n, d)
    w1b = W1.astype(jnp.bfloat16)
    w2b = W2.astype(jnp.bfloat16)

    w1v, w2v, s12, be, nblk, xlo, xhi = _route(xf, Wg, nb)
    s12r = s12.reshape(1, 2 * n)

    xslo, xshi = _sc_dispatch(xlo, xhi, s12r, p)
    ylo, yhi = _grouped_mlp(be.reshape(nb), nblk.reshape(e_num), xslo, xshi,
                            w1b, w2b, b1, b2, nb, p)
    glo, ghi = _sc_combine(ylo, yhi, s12r, n)
    out = _combine(w1v, w2v, glo, ghi)
    return out.reshape(bv, tv, d)


# speedup vs baseline: 1.6944x; 1.0225x over previous
import functools
import jax
import jax.numpy as jnp
from jax.experimental import pallas as pl
from jax.experimental.pallas import tpu as pltpu
from jax.experimental.pallas import tpu_sc as plsc

BLK = 512
RCH = 512
SCW = 128


def _route_kernel(x_ref, wg_ref, w1_ref, w2_ref, s12_ref, be_ref,
                  nblk_ref, xlo_ref, xhi_ref, sel_ref, i1_ref, i2_ref,
                  ltri_ref, *, n, e_num, nb, blk):
    d2 = x_ref.shape[1] // 2
    rch = RCH

    r0 = jax.lax.broadcasted_iota(jnp.int32, (rch, rch), 0)
    r1 = jax.lax.broadcasted_iota(jnp.int32, (rch, rch), 1)
    ltri_ref[...] = jnp.where(r0 > r1, 1.0, 0.0).astype(jnp.bfloat16)

    def p1(r, counts):
        sl = pl.ds(r * rch, rch)
        lg = jax.lax.dot_general(
            x_ref[sl, :], wg_ref[...], (((1,), (1,)), ((), ())),
            preferred_element_type=jnp.float32)
        xlo_ref[sl, :] = x_ref[sl, 0:d2]
        xhi_ref[sl, :] = x_ref[sl, d2:2 * d2]
        iota = jax.lax.broadcasted_iota(jnp.int32, lg.shape, 1)
        m1 = jnp.max(lg, axis=1, keepdims=True)
        i1 = jnp.min(jnp.where(lg == m1, iota, e_num), axis=1, keepdims=True)
        l2 = jnp.where(iota == i1, -jnp.inf, lg)
        m2 = jnp.max(l2, axis=1, keepdims=True)
        i2 = jnp.min(jnp.where(l2 == m2, iota, e_num), axis=1, keepdims=True)
        z = jnp.exp(m2 - m1)
        w1_ref[sl, :] = 1.0 / (1.0 + z)
        w2_ref[sl, :] = z / (1.0 + z)
        sel = (jnp.where(iota == i1, 1.0, 0.0)
               + jnp.where(iota == i2, 1.0, 0.0))
        sel_ref[sl, :] = sel.astype(jnp.bfloat16)
        i1_ref[sl, :] = i1
        i2_ref[sl, :] = i2
        return counts + jnp.sum(sel, axis=0, keepdims=True)

    counts = jax.lax.fori_loop(
        0, n // rch, p1, jnp.zeros((1, e_num), jnp.float32))

    pc = jnp.ceil(counts * (1.0 / blk)) * blk
    pc8 = jnp.broadcast_to(pc, (e_num, e_num))
    i0 = jax.lax.broadcasted_iota(jnp.int32, (e_num, 128), 0)
    i1g = jax.lax.broadcasted_iota(jnp.int32, (e_num, 128), 1)
    upper = jnp.where(i0 < i1g, 1.0, 0.0)
    base128 = jax.lax.dot_general(
        pc8, upper, (((1,), (0,)), ((), ())),
        preferred_element_type=jnp.float32)
    base_row = base128[0:1, 0:e_num]
    c_row = base_row + pc

    nblk_ref[...] = (pc * (1.0 / blk)).astype(jnp.int32)
    c_nb = jnp.broadcast_to(c_row, (nb, e_num))
    b_iota = (jax.lax.broadcasted_iota(jnp.int32, (nb, e_num), 0)
              * blk).astype(jnp.float32)
    be = jnp.sum(jnp.where(c_nb <= b_iota, 1.0, 0.0), axis=1, keepdims=True)
    be_ref[...] = jnp.minimum(be, e_num - 1).astype(jnp.int32)

    def p2(r, run):
        sl = pl.ds(r * rch, rch)
        sel = sel_ref[sl, :]
        i1 = i1_ref[sl, :]
        i2 = i2_ref[sl, :]
        iota = jax.lax.broadcasted_iota(jnp.int32, (rch, e_num), 1)
        pos = jax.lax.dot_general(
            ltri_ref[...], sel, (((1,), (0,)), ((), ())),
            preferred_element_type=jnp.float32) + run
        base_b = jnp.broadcast_to(base_row, (rch, e_num))
        pos1 = jnp.sum(jnp.where(iota == i1, pos + base_b, 0.0),
                       axis=1, keepdims=True)
        pos2 = jnp.sum(jnp.where(iota == i2, pos + base_b, 0.0),
                       axis=1, keepdims=True)
        s12_ref[sl, :] = pos1.astype(jnp.int32)
        s12_ref[pl.ds(n + r * rch, rch), :] = pos2.astype(jnp.int32)
        return run + jnp.sum(sel.astype(jnp.float32), axis=0, keepdims=True)

    jax.lax.fori_loop(0, n // rch, p2, jnp.zeros((1, e_num), jnp.float32))


def _route(x, Wg, nb):
    n, d = x.shape
    e_num = Wg.shape[0]
    kern = functools.partial(_route_kernel, n=n, e_num=e_num, nb=nb, blk=BLK)
    return pl.pallas_call(
        kern,
        grid=(1,),
        in_specs=[
            pl.BlockSpec((n, d), lambda i: (0, 0)),
            pl.BlockSpec((e_num, d), lambda i: (0, 0)),
        ],
        out_specs=[
            pl.BlockSpec((n, 1), lambda i: (0, 0)),
            pl.BlockSpec((n, 1), lambda i: (0, 0)),
            pl.BlockSpec((2 * n, 1), lambda i: (0, 0)),
            pl.BlockSpec((nb, 1), lambda i: (0, 0)),
            pl.BlockSpec((1, e_num), lambda i: (0, 0)),
            pl.BlockSpec((n, d // 2), lambda i: (0, 0)),
            pl.BlockSpec((n, d // 2), lambda i: (0, 0)),
        ],
        out_shape=[
            jax.ShapeDtypeStruct((n, 1), jnp.float32),
            jax.ShapeDtypeStruct((n, 1), jnp.float32),
            jax.ShapeDtypeStruct((2 * n, 1), jnp.int32),
            jax.ShapeDtypeStruct((nb, 1), jnp.int32),
            jax.ShapeDtypeStruct((1, e_num), jnp.int32),
            jax.ShapeDtypeStruct((n, d // 2), jnp.float32),
            jax.ShapeDtypeStruct((n, d // 2), jnp.float32),
        ],
        scratch_shapes=[
            pltpu.VMEM((n, e_num), jnp.bfloat16),
            pltpu.VMEM((n, 1), jnp.int32),
            pltpu.VMEM((n, 1), jnp.int32),
            pltpu.VMEM((RCH, RCH), jnp.bfloat16),
        ],
    )(x, Wg)


def _sc_dispatch(xlo, xhi, s12r, p):
    n2, d2 = 2 * xlo.shape[0], xlo.shape[1]
    nblks = xlo.shape[0] // SCW
    mesh = plsc.VectorSubcoreMesh(core_axis_name="c", subcore_axis_name="s")
    out_type = [jax.ShapeDtypeStruct((p, d2), jnp.float32)] * 2

    @functools.partial(pl.kernel, out_type=out_type, mesh=mesh)
    def k(xlo_hbm, xhi_hbm, s_hbm, olo_hbm, ohi_hbm):
        for x_hbm, o_hbm in ((xlo_hbm, olo_hbm), (xhi_hbm, ohi_hbm)):
            def body(x_vmem, i_vmem, o_hbm=o_hbm):
                pltpu.sync_copy(x_vmem, o_hbm.at[i_vmem.at[0]])

            pltpu.emit_pipeline(
                body,
                grid=(n2 // SCW,),
                in_specs=[
                    pl.BlockSpec((SCW, d2), lambda i: (i % nblks, 0)),
                    pl.BlockSpec((1, SCW), lambda i: (0, i)),
                ],
                out_specs=[],
                core_axis_name=("c", "s"),
                dimension_semantics=(pltpu.PARALLEL,),
            )(x_hbm, s_hbm)

    return k(xlo, xhi, s12r)


def _mlp_kernel(be_ref, nblk_ref, xlo_ref, xhi_ref, w1_ref, w2_ref, b1_ref,
                b2_ref, ylo_ref, yhi_ref):
    d2 = xlo_ref.shape[1]
    b = pl.program_id(0)
    tot = sum(nblk_ref[i] for i in range(nblk_ref.shape[0]))

    @pl.when(b < tot)
    def _():
        xb = jnp.concatenate([xlo_ref[...], xhi_ref[...]],
                             axis=1).astype(jnp.bfloat16)
        h = jax.lax.dot_general(
            xb, w1_ref[0], (((1,), (1,)), ((), ())),
            preferred_element_type=jnp.float32)
        h = h + b1_ref[0]
        h = 0.5 * h * (1.0 + jax.lax.erf(h * 0.7071067811865476))
        hb = h.astype(jnp.bfloat16)
        y = jax.lax.dot_general(
            hb, w2_ref[0], (((1,), (1,)), ((), ())),
            preferred_element_type=jnp.float32)
        y = y + b2_ref[0]
        ylo_ref[...] = y[:, 0:d2]
        yhi_ref[...] = y[:, d2:2 * d2]


def _grouped_mlp(be, nblk8, xslo, xshi, w1b, w2b, b1, b2, nb, p):
    d, f = w2b.shape[1], w2b.shape[2]
    d2 = d // 2
    e_num = w1b.shape[0]
    grid_spec = pltpu.PrefetchScalarGridSpec(
        num_scalar_prefetch=2,
        grid=(nb,),
        in_specs=[
            pl.BlockSpec((BLK, d2), lambda b, be_r, nb_r: (b, 0)),
            pl.BlockSpec((BLK, d2), lambda b, be_r, nb_r: (b, 0)),
            pl.BlockSpec((1, f, d), lambda b, be_r, nb_r: (be_r[b], 0, 0)),
            pl.BlockSpec((1, d, f), lambda b, be_r, nb_r: (be_r[b], 0, 0)),
            pl.BlockSpec((1, 1, f), lambda b, be_r, nb_r: (be_r[b], 0, 0)),
            pl.BlockSpec((1, 1, d), lambda b, be_r, nb_r: (be_r[b], 0, 0)),
        ],
        out_specs=[
            pl.BlockSpec((BLK, d2), lambda b, be_r, nb_r: (b, 0)),
            pl.BlockSpec((BLK, d2), lambda b, be_r, nb_r: (b, 0)),
        ],
    )
    return pl.pallas_call(
        _mlp_kernel,
        grid_spec=grid_spec,
        out_shape=[jax.ShapeDtypeStruct((p, d2), jnp.float32)] * 2,
        compiler_params=pltpu.CompilerParams(
            dimension_semantics=("arbitrary",),
            vmem_limit_bytes=100 * 1024 * 1024,
        ),
    )(be, nblk8, xslo, xshi, w1b, w2b,
      b1.reshape(e_num, 1, f), b2.reshape(e_num, 1, d))


def _sc_combine(ylo, yhi, s12r, n):
    d2 = ylo.shape[1]
    mesh = plsc.VectorSubcoreMesh(core_axis_name="c", subcore_axis_name="s")
    out_type = [jax.ShapeDtypeStruct((2 * n, d2), jnp.float32)] * 2

    @functools.partial(pl.kernel, out_type=out_type, mesh=mesh)
    def k(ylo_hbm, yhi_hbm, s_hbm, glo_hbm, ghi_hbm):
        for y_hbm, g_hbm in ((ylo_hbm, glo_hbm), (yhi_hbm, ghi_hbm)):
            def body(i_vmem, o_vmem, y_hbm=y_hbm):
                pltpu.sync_copy(y_hbm.at[i_vmem.at[0]], o_vmem)

            pltpu.emit_pipeline(
                body,
                grid=(2 * n // SCW,),
                in_specs=[pl.BlockSpec((1, SCW), lambda i: (0, i))],
                out_specs=[pl.BlockSpec((SCW, d2), lambda i: (i, 0))],
                core_axis_name=("c", "s"),
                dimension_semantics=(pltpu.PARALLEL,),
            )(s_hbm, g_hbm)

    return k(ylo, yhi, s12r)


def _comb_kernel(w1_ref, w2_ref, y0lo_ref, y0hi_ref, y1lo_ref, y1hi_ref,
                 o_ref):
    d2 = y0lo_ref.shape[1]
    w1 = w1_ref[...]
    w2 = w2_ref[...]
    o_ref[:, 0:d2] = w1 * y0lo_ref[...] + w2 * y1lo_ref[...]
    o_ref[:, d2:2 * d2] = w1 * y0hi_ref[...] + w2 * y1hi_ref[...]


def _combine(w1v, w2v, glo, ghi):
    n2, d2 = glo.shape
    n = n2 // 2
    cch = 1024
    nch = n // cch
    top = pl.BlockSpec((cch, d2), lambda i: (i, 0))
    bot = pl.BlockSpec((cch, d2), lambda i: (i + nch, 0))
    col = pl.BlockSpec((cch, 1), lambda i: (i, 0))
    return pl.pallas_call(
        _comb_kernel,
        grid=(nch,),
        in_specs=[col, col, top, top, bot, bot],
        out_specs=pl.BlockSpec((cch, 2 * d2), lambda i: (i, 0)),
        out_shape=jax.ShapeDtypeStruct((n, 2 * d2), jnp.float32),
    )(w1v, w2v, glo, ghi, glo, ghi)


def kernel(x, Wg, W1, b1, W2, b2):
    bv, tv, d = x.shape
    n = bv * tv
    e_num, f = W1.shape[0], W1.shape[1]
    nb = 2 * n // BLK + e_num
    p = nb * BLK

    xf = x.reshape(n, d)
    w1b = W1.astype(jnp.bfloat16)
    w2b = W2.astype(jnp.bfloat16)

    w1v, w2v, s12, be, nblk, xlo, xhi = _route(xf, Wg, nb)
    s12r = s12.reshape(1, 2 * n)

    xslo, xshi = _sc_dispatch(xlo, xhi, s12r, p)
    ylo, yhi = _grouped_mlp(be.reshape(nb), nblk.reshape(e_num), xslo, xshi,
                            w1b, w2b, b1, b2, nb, p)
    glo, ghi = _sc_combine(ylo, yhi, s12r, n)
    out = _combine(w1v, w2v, glo, ghi)
    return out.reshape(bv, tv, d)
